# baseline (speedup 1.0000x reference)
import jax
import jax.numpy as jnp
from jax.experimental import pallas as pl

_BLOCK = 2048


def _rev_block(in_ref, out_ref):
    b = in_ref.shape[0]
    rows = jax.lax.broadcasted_iota(jnp.int32, (b, b), 0)
    cols = jax.lax.broadcasted_iota(jnp.int32, (b, b), 1)
    perm = (rows + cols == b - 1).astype(jnp.float32)
    out_ref[:] = jnp.dot(perm, in_ref[:], preferred_element_type=jnp.float32)


def kernel(x, pos_emb):
    maxlen = x.shape[-1]
    dim = pos_emb.shape[1]
    num_blocks = maxlen // _BLOCK
    return pl.pallas_call(
        _rev_block,
        grid=(num_blocks,),
        in_specs=[
            pl.BlockSpec((_BLOCK, dim), lambda i: (num_blocks - 1 - i, 0)),
        ],
        out_specs=pl.BlockSpec((_BLOCK, dim), lambda i: (i, 0)),
        out_shape=jax.ShapeDtypeStruct((maxlen, dim), pos_emb.dtype),
    )(pos_emb[:maxlen])


# speedup vs baseline: 1.5204x; 1.5204x over previous
import jax
import jax.numpy as jnp
from jax.experimental import pallas as pl

_BLOCK = 1024
_CHUNK = 128


def _rev_block(in_ref, out_ref):
    b = in_ref.shape[0]
    rows = jax.lax.broadcasted_iota(jnp.int32, (_CHUNK, _CHUNK), 0)
    cols = jax.lax.broadcasted_iota(jnp.int32, (_CHUNK, _CHUNK), 1)
    perm = (rows + cols == _CHUNK - 1).astype(jnp.float32)
    n = b // _CHUNK
    for k in range(n):
        src = (n - 1 - k) * _CHUNK
        out_ref[k * _CHUNK:(k + 1) * _CHUNK, :] = jnp.dot(
            perm, in_ref[src:src + _CHUNK, :],
            preferred_element_type=jnp.float32)


def kernel(x, pos_emb):
    maxlen = x.shape[-1]
    dim = pos_emb.shape[1]
    num_blocks = maxlen // _BLOCK
    return pl.pallas_call(
        _rev_block,
        grid=(num_blocks,),
        in_specs=[
            pl.BlockSpec((_BLOCK, dim), lambda i: (num_blocks - 1 - i, 0)),
        ],
        out_specs=pl.BlockSpec((_BLOCK, dim), lambda i: (i, 0)),
        out_shape=jax.ShapeDtypeStruct((maxlen, dim), pos_emb.dtype),
    )(pos_emb[:maxlen])


# speedup vs baseline: 2.1707x; 1.4277x over previous
import jax
import jax.numpy as jnp
from jax.experimental import pallas as pl

_BLOCK = 2048
_CHUNK = 128


def _rev_block(in_ref, out_ref):
    b = in_ref.shape[0]
    rows = jax.lax.broadcasted_iota(jnp.int32, (_CHUNK, _CHUNK), 0)
    cols = jax.lax.broadcasted_iota(jnp.int32, (_CHUNK, _CHUNK), 1)
    perm = (rows + cols == _CHUNK - 1).astype(jnp.float32)
    n = b // _CHUNK
    for k in range(n):
        src = (n - 1 - k) * _CHUNK
        out_ref[k * _CHUNK:(k + 1) * _CHUNK, :] = jnp.dot(
            perm, in_ref[src:src + _CHUNK, :],
            preferred_element_type=jnp.float32)


def kernel(x, pos_emb):
    maxlen = x.shape[-1]
    dim = pos_emb.shape[1]
    num_blocks = maxlen // _BLOCK
    return pl.pallas_call(
        _rev_block,
        grid=(num_blocks,),
        in_specs=[
            pl.BlockSpec((_BLOCK, dim), lambda i: (num_blocks - 1 - i, 0)),
        ],
        out_specs=pl.BlockSpec((_BLOCK, dim), lambda i: (i, 0)),
        out_shape=jax.ShapeDtypeStruct((maxlen, dim), pos_emb.dtype),
    )(pos_emb[:maxlen])


# speedup vs baseline: 2.8624x; 1.3187x over previous
import jax
import jax.numpy as jnp
from jax.experimental import pallas as pl

_BLOCK = 4096
_CHUNK = 128


def _rev_block(in_ref, out_ref):
    b = in_ref.shape[0]
    rows = jax.lax.broadcasted_iota(jnp.int32, (_CHUNK, _CHUNK), 0)
    cols = jax.lax.broadcasted_iota(jnp.int32, (_CHUNK, _CHUNK), 1)
    perm = (rows + cols == _CHUNK - 1).astype(jnp.float32)
    n = b // _CHUNK
    for k in range(n):
        src = (n - 1 - k) * _CHUNK
        out_ref[k * _CHUNK:(k + 1) * _CHUNK, :] = jnp.dot(
            perm, in_ref[src:src + _CHUNK, :],
            preferred_element_type=jnp.float32)


def kernel(x, pos_emb):
    maxlen = x.shape[-1]
    dim = pos_emb.shape[1]
    num_blocks = maxlen // _BLOCK
    return pl.pallas_call(
        _rev_block,
        grid=(num_blocks,),
        in_specs=[
            pl.BlockSpec((_BLOCK, dim), lambda i: (num_blocks - 1 - i, 0)),
        ],
        out_specs=pl.BlockSpec((_BLOCK, dim), lambda i: (i, 0)),
        out_shape=jax.ShapeDtypeStruct((maxlen, dim), pos_emb.dtype),
    )(pos_emb[:maxlen])
